# baseline (speedup 1.0000x reference)
import functools

import jax
import jax.numpy as jnp
from jax import lax
from jax.experimental import pallas as pl
from jax.experimental.pallas import tpu as pltpu
from jax.experimental.pallas import tpu_sc as plsc

VOCAB = 1000000
SEQ = 200
EMBED = 64
BATCH = 4096

NC = 2
NS = 16
NW = NC * NS
SEQS_PER_W = BATCH // NW
ROWS_PER_W = SEQS_PER_W * SEQ
LANES = 16
VPR = EMBED // LANES

_mesh = plsc.VectorSubcoreMesh(core_axis_name="c", subcore_axis_name="s")


@functools.partial(
    pl.kernel,
    out_type=jax.ShapeDtypeStruct((BATCH * SEQ, EMBED), jnp.float32),
    mesh=_mesh,
    compiler_params=pltpu.CompilerParams(use_tc_tiling_on_sc=False),
    scratch_types=[
        pltpu.VMEM((ROWS_PER_W,), jnp.int32),
        pltpu.VMEM((SEQ, EMBED), jnp.float32),
        pltpu.VMEM((SEQ, EMBED), jnp.float32),
        pltpu.SemaphoreType.DMA,
    ],
)
def _seq_embed(seq_hbm, tok_hbm, pos_hbm, out_hbm, idx_v, pos_v, buf, sem):
    wid = lax.axis_index("s") * NC + lax.axis_index("c")
    base = wid * ROWS_PER_W

    pltpu.sync_copy(seq_hbm.at[pl.ds(base, ROWS_PER_W)], idx_v)
    pltpu.sync_copy(pos_hbm, pos_v)

    def chunk(c, carry):
        row0 = c * SEQ
        g0 = pltpu.async_copy(
            tok_hbm.at[idx_v.at[pl.ds(row0, 128)]], buf.at[pl.ds(0, 128)], sem)
        g1 = pltpu.async_copy(
            tok_hbm.at[idx_v.at[pl.ds(row0 + 128, SEQ - 128)]],
            buf.at[pl.ds(128, SEQ - 128)], sem)
        g0.wait()
        g1.wait()

        def add_row(j, carry2):
            for k in range(VPR):
                plsc.addupdate(
                    buf.at[j, pl.ds(k * LANES, LANES)],
                    pos_v[j, pl.ds(k * LANES, LANES)])
            return carry2

        lax.fori_loop(0, SEQ, add_row, 0, unroll=2)

        pltpu.sync_copy(buf, out_hbm.at[pl.ds(base + row0, SEQ)])
        return carry

    lax.fori_loop(0, SEQS_PER_W, chunk, 0)


def kernel(sequence, token_table, pos_table):
    seq_flat = sequence.reshape(-1).astype(jnp.int32)
    out = _seq_embed(seq_flat, token_table, pos_table)
    return out.reshape(BATCH, SEQ, EMBED)
